# baseline (speedup 1.0000x reference)
import functools

import jax
import jax.numpy as jnp
from jax import lax
from jax.experimental import pallas as pl
from jax.experimental.pallas import tpu as pltpu
from jax.experimental.pallas import tpu_sc as plsc

N = 10000
E = 320000
D_IN = 128
D_HID = 64

NC = 2
NS = 16
NW = NC * NS
CHUNK = 128
EROWS = E // CHUNK
ROWS_BASE = EROWS // NW
ROWS_EXTRA = EROWS - ROWS_BASE * NW
NCHUNK3 = ROWS_BASE // 3
ROWS_PER_TILE = 624
ROWS_TAIL = N - ROWS_PER_TILE * NS
DEG_W = 16
RB = ROWS_PER_TILE // 16


def _mm1_body(x_ref, w_ref, b_ref, o_ref):
    o_ref[...] = (
        jnp.dot(x_ref[...], w_ref[...], preferred_element_type=jnp.float32)
        + b_ref[...]
    )


def _stage1(x, W1, b1):
    B = 1000
    x2 = x.reshape(N // 2, 2 * D_IN)
    wbd = jnp.zeros((2 * D_IN, 2 * D_HID), jnp.float32)
    wbd = wbd.at[:D_IN, :D_HID].set(W1).at[D_IN:, D_HID:].set(W1)
    bbd = jnp.concatenate([b1, b1]).reshape(1, 2 * D_HID)
    h2 = pl.pallas_call(
        _mm1_body,
        grid=(N // 2 // B,),
        in_specs=[
            pl.BlockSpec((B, 2 * D_IN), lambda i: (i, 0)),
            pl.BlockSpec((2 * D_IN, 2 * D_HID), lambda i: (0, 0)),
            pl.BlockSpec((1, 2 * D_HID), lambda i: (0, 0)),
        ],
        out_specs=pl.BlockSpec((B, 2 * D_HID), lambda i: (i, 0)),
        out_shape=jax.ShapeDtypeStruct((N // 2, 2 * D_HID), jnp.float32),
    )(x2, wbd, bbd)
    return h2.reshape(N, D_HID)


def _sc_deg_body(edges_hbm, z16_hbm, ones_hbm, deg_out,
                 deg_sh, dst_v, ones_v, sem_a, sem_b):
    cid = lax.axis_index("c")
    sid = lax.axis_index("s")
    wid = sid * NC + cid

    rbase = sid * ROWS_PER_TILE
    pltpu.sync_copy(z16_hbm.at[pl.ds(rbase, ROWS_PER_TILE)],
                    deg_sh.at[pl.ds(rbase, ROWS_PER_TILE)])

    @pl.when(sid == NS - 1)
    def _zero_tail():
        tb = ROWS_PER_TILE * NS
        pltpu.sync_copy(z16_hbm.at[pl.ds(tb, ROWS_TAIL)],
                        deg_sh.at[pl.ds(tb, ROWS_TAIL)])

    row_off = ROWS_BASE * wid + jnp.minimum(wid, ROWS_EXTRA)
    pltpu.sync_copy(edges_hbm.at[1, pl.ds(row_off, ROWS_BASE)],
                    dst_v.at[pl.ds(0, ROWS_BASE)])

    @pl.when(wid < ROWS_EXTRA)
    def _stage_extra():
        pltpu.sync_copy(edges_hbm.at[1, pl.ds(row_off + ROWS_BASE, 1)],
                        dst_v.at[pl.ds(ROWS_BASE, 1)])

    pltpu.sync_copy(ones_hbm, ones_v)
    plsc.subcore_barrier()

    def _fire(j, sem):
        pltpu.async_copy(ones_v, deg_sh.at[dst_v.at[j]], sem, add=True)

    def _wait(j, sem):
        pltpu.make_async_copy(ones_v, deg_sh.at[dst_v.at[j]], sem).wait()

    _fire(0, sem_a)

    def body(jj, carry):
        j0 = 2 * jj
        _fire(j0 + 1, sem_b)
        _wait(j0, sem_a)

        @pl.when(jj + 1 < ROWS_BASE // 2)
        def _():
            _fire(j0 + 2, sem_a)

        _wait(j0 + 1, sem_b)
        return carry

    lax.fori_loop(0, ROWS_BASE // 2, body, 0)

    @pl.when(wid < ROWS_EXTRA)
    def _extra_chunk():
        _fire(ROWS_BASE, sem_a)
        _wait(ROWS_BASE, sem_a)

    plsc.subcore_barrier()

    pltpu.sync_copy(deg_sh.at[pl.ds(rbase, ROWS_PER_TILE)],
                    deg_out.at[pl.ds(rbase, ROWS_PER_TILE),
                               pl.ds(cid * DEG_W, DEG_W)])

    @pl.when(sid == NS - 1)
    def _publish_tail():
        tb = ROWS_PER_TILE * NS
        pltpu.sync_copy(deg_sh.at[pl.ds(tb, ROWS_TAIL)],
                        deg_out.at[pl.ds(tb, ROWS_TAIL),
                                   pl.ds(cid * DEG_W, DEG_W)])


def _sc_deg(edges3):
    z16 = jnp.zeros((N, DEG_W), jnp.float32)
    ones = jnp.ones((CHUNK, DEG_W), jnp.float32)
    mesh = plsc.VectorSubcoreMesh(core_axis_name="c", subcore_axis_name="s")
    f = functools.partial(
        pl.kernel,
        out_type=jax.ShapeDtypeStruct((N, 8 * DEG_W), jnp.float32),
        mesh=mesh,
        compiler_params=pltpu.CompilerParams(use_tc_tiling_on_sc=False),
        scratch_types=[
            pltpu.VMEM_SHARED((N, DEG_W), jnp.float32),
            pltpu.VMEM((ROWS_BASE + 1, CHUNK), jnp.int32),
            pltpu.VMEM((CHUNK, DEG_W), jnp.float32),
            pltpu.SemaphoreType.DMA,
            pltpu.SemaphoreType.DMA,
        ],
    )(_sc_deg_body)
    return f(edges3, z16, ones)


def _sc_agg_body(h_hbm, edges_hbm, z64_hbm,
                 agg_out,
                 agg_sh, src_v, dst_v, rows0, rows1, rows2,
                 sem_g0, sem_g1, sem_g2, sem_s0, sem_s1, sem_s2):
    cid = lax.axis_index("c")
    sid = lax.axis_index("s")
    wid = sid * NC + cid

    rbase = sid * ROWS_PER_TILE
    pltpu.sync_copy(z64_hbm.at[pl.ds(rbase, ROWS_PER_TILE)],
                    agg_sh.at[pl.ds(rbase, ROWS_PER_TILE)])

    @pl.when(sid == NS - 1)
    def _zero_tail():
        tb = ROWS_PER_TILE * NS
        pltpu.sync_copy(z64_hbm.at[pl.ds(tb, ROWS_TAIL)],
                        agg_sh.at[pl.ds(tb, ROWS_TAIL)])

    row_off = ROWS_BASE * wid + jnp.minimum(wid, ROWS_EXTRA)
    pltpu.sync_copy(edges_hbm.at[0, pl.ds(row_off, ROWS_BASE)],
                    src_v.at[pl.ds(0, ROWS_BASE)])
    pltpu.sync_copy(edges_hbm.at[1, pl.ds(row_off, ROWS_BASE)],
                    dst_v.at[pl.ds(0, ROWS_BASE)])

    @pl.when(wid < ROWS_EXTRA)
    def _stage_extra():
        pltpu.sync_copy(edges_hbm.at[0, pl.ds(row_off + ROWS_BASE, 1)],
                        src_v.at[pl.ds(ROWS_BASE, 1)])
        pltpu.sync_copy(edges_hbm.at[1, pl.ds(row_off + ROWS_BASE, 1)],
                        dst_v.at[pl.ds(ROWS_BASE, 1)])

    plsc.subcore_barrier()

    def _fire_gather(j, buf, sem):
        pltpu.async_copy(h_hbm.at[src_v.at[j]], buf, sem)

    def _wait_gather(j, buf, sem):
        pltpu.make_async_copy(h_hbm.at[src_v.at[j]], buf, sem).wait()

    def _fire_scatter(j, buf, sem):
        pltpu.async_copy(buf, agg_sh.at[dst_v.at[j]], sem, add=True)

    def _wait_scatter(j, buf, sem):
        pltpu.make_async_copy(buf, agg_sh.at[dst_v.at[j]], sem).wait()

    _fire_gather(0, rows0, sem_g0)

    def body(jj, carry):
        j0 = 3 * jj

        @pl.when(jj > 0)
        def _():
            _wait_scatter(j0 - 2, rows1, sem_s1)

        _fire_gather(j0 + 1, rows1, sem_g1)
        _wait_gather(j0, rows0, sem_g0)
        _fire_scatter(j0, rows0, sem_s0)

        @pl.when(jj > 0)
        def _():
            _wait_scatter(j0 - 1, rows2, sem_s2)

        _fire_gather(j0 + 2, rows2, sem_g2)
        _wait_gather(j0 + 1, rows1, sem_g1)
        _fire_scatter(j0 + 1, rows1, sem_s1)
        _wait_scatter(j0, rows0, sem_s0)

        @pl.when(jj + 1 < NCHUNK3)
        def _():
            _fire_gather(j0 + 3, rows0, sem_g0)

        _wait_gather(j0 + 2, rows2, sem_g2)
        _fire_scatter(j0 + 2, rows2, sem_s2)
        return carry

    lax.fori_loop(0, NCHUNK3, body, 0)
    _wait_scatter(ROWS_BASE - 2, rows1, sem_s1)
    _wait_scatter(ROWS_BASE - 1, rows2, sem_s2)

    @pl.when(wid < ROWS_EXTRA)
    def _extra_chunk():
        _fire_gather(ROWS_BASE, rows0, sem_g0)
        _wait_gather(ROWS_BASE, rows0, sem_g0)
        _fire_scatter(ROWS_BASE, rows0, sem_s0)
        _wait_scatter(ROWS_BASE, rows0, sem_s0)

    plsc.subcore_barrier()

    pltpu.sync_copy(agg_sh.at[pl.ds(rbase, ROWS_PER_TILE)],
                    agg_out.at[pl.ds(rbase, ROWS_PER_TILE),
                               pl.ds(cid * D_HID, D_HID)])

    @pl.when(sid == NS - 1)
    def _publish_tail():
        tb = ROWS_PER_TILE * NS
        pltpu.sync_copy(agg_sh.at[pl.ds(tb, ROWS_TAIL)],
                        agg_out.at[pl.ds(tb, ROWS_TAIL),
                                   pl.ds(cid * D_HID, D_HID)])


def _sc_agg(h, edges3):
    z64 = jnp.zeros((N, D_HID), jnp.float32)
    mesh = plsc.VectorSubcoreMesh(core_axis_name="c", subcore_axis_name="s")
    f = functools.partial(
        pl.kernel,
        out_type=jax.ShapeDtypeStruct((N, 2 * D_HID), jnp.float32),
        mesh=mesh,
        compiler_params=pltpu.CompilerParams(use_tc_tiling_on_sc=False),
        scratch_types=[
            pltpu.VMEM_SHARED((N, D_HID), jnp.float32),
            pltpu.VMEM((ROWS_BASE + 1, CHUNK), jnp.int32),
            pltpu.VMEM((ROWS_BASE + 1, CHUNK), jnp.int32),
            pltpu.VMEM((CHUNK, D_HID), jnp.float32),
            pltpu.VMEM((CHUNK, D_HID), jnp.float32),
            pltpu.VMEM((CHUNK, D_HID), jnp.float32),
            pltpu.SemaphoreType.DMA,
            pltpu.SemaphoreType.DMA,
            pltpu.SemaphoreType.DMA,
            pltpu.SemaphoreType.DMA,
            pltpu.SemaphoreType.DMA,
            pltpu.SemaphoreType.DMA,
        ],
    )(_sc_agg_body)
    return f(h, edges3, z64)


def _fin_body(agg_ref, deg_ref, w_ref, b_ref, a_ref, o_ref):
    s = agg_ref[:, :D_HID] + agg_ref[:, D_HID:]
    d = deg_ref[:, 0:1] + deg_ref[:, DEG_W:DEG_W + 1]
    m = s / jnp.maximum(d, 1.0)
    p = jnp.where(m >= 0, m, a_ref[...] * m)
    o_ref[...] = (
        jnp.dot(p, w_ref[...], preferred_element_type=jnp.float32) + b_ref[...]
    )


def _stage3(aggc, degp, W2, b2, a):
    B = 2000
    a_row = jnp.full((1, D_HID), a, jnp.float32)
    return pl.pallas_call(
        _fin_body,
        grid=(N // B,),
        in_specs=[
            pl.BlockSpec((B, 2 * D_HID), lambda i: (i, 0)),
            pl.BlockSpec((B, 8 * DEG_W), lambda i: (i, 0)),
            pl.BlockSpec((D_HID, D_HID), lambda i: (0, 0)),
            pl.BlockSpec((1, D_HID), lambda i: (0, 0)),
            pl.BlockSpec((1, D_HID), lambda i: (0, 0)),
        ],
        out_specs=pl.BlockSpec((B, D_HID), lambda i: (i, 0)),
        out_shape=jax.ShapeDtypeStruct((N, D_HID), jnp.float32),
    )(aggc, degp, W2, b2.reshape(1, D_HID), a_row)


def kernel(x, edge_index, W1, b1, W2, b2, a):
    edges3 = edge_index.reshape(2, EROWS, CHUNK)
    degp = _sc_deg(edges3)
    h = _stage1(x, W1, b1)
    aggc = _sc_agg(h, edges3)
    return _stage3(aggc, degp, W2, b2, a)


# speedup vs baseline: 1.0953x; 1.0953x over previous
import functools

import jax
import jax.numpy as jnp
from jax import lax
from jax.experimental import pallas as pl
from jax.experimental.pallas import tpu as pltpu
from jax.experimental.pallas import tpu_sc as plsc

N = 10000
E = 320000
D_IN = 128
D_HID = 64

NC = 2
NS = 16
NW = NC * NS
CHUNK = 128
EROWS = E // CHUNK
ROWS_BASE = EROWS // NW
ROWS_EXTRA = EROWS - ROWS_BASE * NW
NCHUNK3 = ROWS_BASE // 3
ROWS_PER_TILE = 624
ROWS_TAIL = N - ROWS_PER_TILE * NS
DEG_W = 16
RB = ROWS_PER_TILE // 16


def _mm1_body(x_ref, w_ref, b_ref, o_ref):
    o_ref[...] = (
        jnp.dot(x_ref[...], w_ref[...], preferred_element_type=jnp.float32)
        + b_ref[...]
    )


def _stage1(x, W1, b1):
    B = 1000
    x2 = x.reshape(N // 2, 2 * D_IN)
    wbd = jnp.zeros((2 * D_IN, 2 * D_HID), jnp.float32)
    wbd = wbd.at[:D_IN, :D_HID].set(W1).at[D_IN:, D_HID:].set(W1)
    bbd = jnp.concatenate([b1, b1]).reshape(1, 2 * D_HID)
    h2 = pl.pallas_call(
        _mm1_body,
        grid=(N // 2 // B,),
        in_specs=[
            pl.BlockSpec((B, 2 * D_IN), lambda i: (i, 0)),
            pl.BlockSpec((2 * D_IN, 2 * D_HID), lambda i: (0, 0)),
            pl.BlockSpec((1, 2 * D_HID), lambda i: (0, 0)),
        ],
        out_specs=pl.BlockSpec((B, 2 * D_HID), lambda i: (i, 0)),
        out_shape=jax.ShapeDtypeStruct((N // 2, 2 * D_HID), jnp.float32),
    )(x2, wbd, bbd)
    return h2.reshape(N, D_HID)


def _sc_deg_body(edges_hbm, z16_hbm, ones_hbm, deg_out,
                 deg_sh, dst_v, ones_v, sem_a, sem_b):
    cid = lax.axis_index("c")
    sid = lax.axis_index("s")
    wid = sid * NC + cid

    rbase = sid * ROWS_PER_TILE
    pltpu.sync_copy(z16_hbm.at[pl.ds(rbase, ROWS_PER_TILE)],
                    deg_sh.at[pl.ds(rbase, ROWS_PER_TILE)])

    @pl.when(sid == NS - 1)
    def _zero_tail():
        tb = ROWS_PER_TILE * NS
        pltpu.sync_copy(z16_hbm.at[pl.ds(tb, ROWS_TAIL)],
                        deg_sh.at[pl.ds(tb, ROWS_TAIL)])

    row_off = ROWS_BASE * wid + jnp.minimum(wid, ROWS_EXTRA)
    pltpu.sync_copy(edges_hbm.at[1, pl.ds(row_off, ROWS_BASE)],
                    dst_v.at[pl.ds(0, ROWS_BASE)])

    @pl.when(wid < ROWS_EXTRA)
    def _stage_extra():
        pltpu.sync_copy(edges_hbm.at[1, pl.ds(row_off + ROWS_BASE, 1)],
                        dst_v.at[pl.ds(ROWS_BASE, 1)])

    pltpu.sync_copy(ones_hbm, ones_v)
    plsc.subcore_barrier()

    def _fire(j, sem):
        pltpu.async_copy(ones_v, deg_sh.at[dst_v.at[j]], sem, add=True)

    def _wait(j, sem):
        pltpu.make_async_copy(ones_v, deg_sh.at[dst_v.at[j]], sem).wait()

    _fire(0, sem_a)

    def body(jj, carry):
        j0 = 2 * jj
        _fire(j0 + 1, sem_b)
        _wait(j0, sem_a)

        @pl.when(jj + 1 < ROWS_BASE // 2)
        def _():
            _fire(j0 + 2, sem_a)

        _wait(j0 + 1, sem_b)
        return carry

    lax.fori_loop(0, ROWS_BASE // 2, body, 0)

    @pl.when(wid < ROWS_EXTRA)
    def _extra_chunk():
        _fire(ROWS_BASE, sem_a)
        _wait(ROWS_BASE, sem_a)

    plsc.subcore_barrier()

    pltpu.sync_copy(deg_sh.at[pl.ds(rbase, ROWS_PER_TILE)],
                    deg_out.at[pl.ds(rbase, ROWS_PER_TILE),
                               pl.ds(cid * DEG_W, DEG_W)])

    @pl.when(sid == NS - 1)
    def _publish_tail():
        tb = ROWS_PER_TILE * NS
        pltpu.sync_copy(deg_sh.at[pl.ds(tb, ROWS_TAIL)],
                        deg_out.at[pl.ds(tb, ROWS_TAIL),
                                   pl.ds(cid * DEG_W, DEG_W)])


def _sc_deg(edges3):
    z16 = jnp.zeros((N, DEG_W), jnp.float32)
    ones = jnp.ones((CHUNK, DEG_W), jnp.float32)
    mesh = plsc.VectorSubcoreMesh(core_axis_name="c", subcore_axis_name="s")
    f = functools.partial(
        pl.kernel,
        out_type=jax.ShapeDtypeStruct((N, 8 * DEG_W), jnp.float32),
        mesh=mesh,
        compiler_params=pltpu.CompilerParams(use_tc_tiling_on_sc=False),
        scratch_types=[
            pltpu.VMEM_SHARED((N, DEG_W), jnp.float32),
            pltpu.VMEM((ROWS_BASE + 1, CHUNK), jnp.int32),
            pltpu.VMEM((CHUNK, DEG_W), jnp.float32),
            pltpu.SemaphoreType.DMA,
            pltpu.SemaphoreType.DMA,
        ],
    )(_sc_deg_body)
    return f(edges3, z16, ones)


def _sc_agg_body(h_hbm, edges_hbm, z64_hbm,
                 agg_out,
                 agg_sh, src_v, dst_v, rows0, rows1, rows2,
                 sem_g0, sem_g1, sem_g2, sem_s0, sem_s1, sem_s2):
    cid = lax.axis_index("c")
    sid = lax.axis_index("s")
    wid = sid * NC + cid

    rbase = sid * ROWS_PER_TILE
    pltpu.sync_copy(z64_hbm.at[pl.ds(rbase, ROWS_PER_TILE)],
                    agg_sh.at[pl.ds(rbase, ROWS_PER_TILE)])

    @pl.when(sid == NS - 1)
    def _zero_tail():
        tb = ROWS_PER_TILE * NS
        pltpu.sync_copy(z64_hbm.at[pl.ds(tb, ROWS_TAIL)],
                        agg_sh.at[pl.ds(tb, ROWS_TAIL)])

    row_off = ROWS_BASE * wid + jnp.minimum(wid, ROWS_EXTRA)
    pltpu.sync_copy(edges_hbm.at[0, pl.ds(row_off, ROWS_BASE)],
                    src_v.at[pl.ds(0, ROWS_BASE)])
    pltpu.sync_copy(edges_hbm.at[1, pl.ds(row_off, ROWS_BASE)],
                    dst_v.at[pl.ds(0, ROWS_BASE)])

    @pl.when(wid < ROWS_EXTRA)
    def _stage_extra():
        pltpu.sync_copy(edges_hbm.at[0, pl.ds(row_off + ROWS_BASE, 1)],
                        src_v.at[pl.ds(ROWS_BASE, 1)])
        pltpu.sync_copy(edges_hbm.at[1, pl.ds(row_off + ROWS_BASE, 1)],
                        dst_v.at[pl.ds(ROWS_BASE, 1)])

    plsc.subcore_barrier()

    def _fire_gather(j, buf, sem):
        pltpu.async_copy(h_hbm.at[src_v.at[j]], buf, sem)

    def _wait_gather(j, buf, sem):
        pltpu.make_async_copy(h_hbm.at[src_v.at[j]], buf, sem).wait()

    def _fire_scatter(j, buf, sem):
        pltpu.async_copy(buf, agg_sh.at[dst_v.at[j]], sem, add=True)

    def _wait_scatter(j, buf, sem):
        pltpu.make_async_copy(buf, agg_sh.at[dst_v.at[j]], sem).wait()

    _fire_gather(0, rows0, sem_g0)

    def body(jj, carry):
        j0 = 3 * jj

        @pl.when(jj > 0)
        def _():
            _wait_scatter(j0 - 2, rows1, sem_s1)

        _fire_gather(j0 + 1, rows1, sem_g1)
        _wait_gather(j0, rows0, sem_g0)
        _fire_scatter(j0, rows0, sem_s0)

        @pl.when(jj > 0)
        def _():
            _wait_scatter(j0 - 1, rows2, sem_s2)

        _fire_gather(j0 + 2, rows2, sem_g2)
        _wait_gather(j0 + 1, rows1, sem_g1)
        _fire_scatter(j0 + 1, rows1, sem_s1)
        _wait_scatter(j0, rows0, sem_s0)

        @pl.when(jj + 1 < NCHUNK3)
        def _():
            _fire_gather(j0 + 3, rows0, sem_g0)

        _wait_gather(j0 + 2, rows2, sem_g2)
        _fire_scatter(j0 + 2, rows2, sem_s2)
        return carry

    lax.fori_loop(0, NCHUNK3, body, 0)
    _wait_scatter(ROWS_BASE - 2, rows1, sem_s1)
    _wait_scatter(ROWS_BASE - 1, rows2, sem_s2)

    @pl.when(wid < ROWS_EXTRA)
    def _extra_chunk():
        _fire_gather(ROWS_BASE, rows0, sem_g0)
        _wait_gather(ROWS_BASE, rows0, sem_g0)
        _fire_scatter(ROWS_BASE, rows0, sem_s0)
        _wait_scatter(ROWS_BASE, rows0, sem_s0)

    plsc.subcore_barrier()

    pltpu.sync_copy(agg_sh.at[pl.ds(rbase, ROWS_PER_TILE)],
                    agg_out.at[pl.ds(rbase, ROWS_PER_TILE),
                               pl.ds(cid * D_HID, D_HID)])

    @pl.when(sid == NS - 1)
    def _publish_tail():
        tb = ROWS_PER_TILE * NS
        pltpu.sync_copy(agg_sh.at[pl.ds(tb, ROWS_TAIL)],
                        agg_out.at[pl.ds(tb, ROWS_TAIL),
                                   pl.ds(cid * D_HID, D_HID)])


def _sc_agg(h, edges3):
    z64 = jnp.zeros((N, D_HID), jnp.float32)
    mesh = plsc.VectorSubcoreMesh(core_axis_name="c", subcore_axis_name="s")
    f = functools.partial(
        pl.kernel,
        out_type=jax.ShapeDtypeStruct((N, 2 * D_HID), jnp.float32),
        mesh=mesh,
        compiler_params=pltpu.CompilerParams(use_tc_tiling_on_sc=False),
        scratch_types=[
            pltpu.VMEM_SHARED((N, D_HID), jnp.float32),
            pltpu.VMEM((ROWS_BASE + 1, CHUNK), jnp.int32),
            pltpu.VMEM((ROWS_BASE + 1, CHUNK), jnp.int32),
            pltpu.VMEM((CHUNK, D_HID), jnp.float32),
            pltpu.VMEM((CHUNK, D_HID), jnp.float32),
            pltpu.VMEM((CHUNK, D_HID), jnp.float32),
            pltpu.SemaphoreType.DMA,
            pltpu.SemaphoreType.DMA,
            pltpu.SemaphoreType.DMA,
            pltpu.SemaphoreType.DMA,
            pltpu.SemaphoreType.DMA,
            pltpu.SemaphoreType.DMA,
        ],
    )(_sc_agg_body)
    return f(h, edges3, z64)


def _fin_body(agg_ref, deg_ref, w_ref, b_ref, a_ref, o_ref):
    s = agg_ref[:, :D_HID] + agg_ref[:, D_HID:]
    d = deg_ref[:, 0:1] + deg_ref[:, DEG_W:DEG_W + 1]
    m = s / jnp.maximum(d, 1.0)
    p = jnp.where(m >= 0, m, a_ref[...] * m)
    o_ref[...] = (
        jnp.dot(p, w_ref[...], preferred_element_type=jnp.float32) + b_ref[...]
    )


def _stage3(aggc, degp, W2, b2, a):
    B = 2000
    a_row = jnp.full((1, D_HID), a, jnp.float32)
    return pl.pallas_call(
        _fin_body,
        grid=(N // B,),
        in_specs=[
            pl.BlockSpec((B, 2 * D_HID), lambda i: (i, 0)),
            pl.BlockSpec((B, 8 * DEG_W), lambda i: (i, 0)),
            pl.BlockSpec((D_HID, D_HID), lambda i: (0, 0)),
            pl.BlockSpec((1, D_HID), lambda i: (0, 0)),
            pl.BlockSpec((1, D_HID), lambda i: (0, 0)),
        ],
        out_specs=pl.BlockSpec((B, D_HID), lambda i: (i, 0)),
        out_shape=jax.ShapeDtypeStruct((N, D_HID), jnp.float32),
    )(aggc, degp, W2, b2.reshape(1, D_HID), a_row)


def kernel(x, edge_index, W1, b1, W2, b2, a):
    edges3 = edge_index.reshape(2, EROWS, CHUNK)
    degp = _sc_deg(edges3)
    h = _stage1(x, W1, b1)
    h, degp = lax.optimization_barrier((h, degp))
    aggc = _sc_agg(h, edges3)
    return _stage3(aggc, degp, W2, b2, a)
